# Initial kernel scaffold; baseline (speedup 1.0000x reference)
import jax
import jax.numpy as jnp
from jax.experimental import pallas as pl


def kernel(probas, labels):
    raise NotImplementedError("write your pallas kernel here")


# speedup vs baseline: 44.2332x; 44.2332x over previous
import functools

import jax
import jax.numpy as jnp
from jax import lax
from jax.experimental import pallas as pl
from jax.experimental.pallas import tpu as pltpu
from jax.experimental.pallas import tpu_sc as plsc

NB = 1024
NB2 = 2 * NB
L = 16
NC = 2
NS = 16
NW = NC * NS
NUM_C = 21
PIX = 512 * 512
CHUNK = PIX // NW
NBATCH = 4


def _stage1_body(p_hbm, lab_hbm, out_hbm, lab_v, p_v, hist_v):
    cid = lax.axis_index("c")
    sid = lax.axis_index("s")
    wid = sid * NC + cid
    base = wid * CHUNK

    pltpu.sync_copy(lab_hbm.at[pl.ds(0, NBATCH), pl.ds(base, CHUNK)], lab_v)

    ones = jnp.ones((L,), jnp.float32)
    fgoff = jnp.full((L,), NB, jnp.int32)
    zoff = jnp.zeros((L,), jnp.int32)
    zvec = jnp.zeros((L,), jnp.float32)
    nbf = jnp.float32(NB)
    nbm1 = jnp.full((L,), NB - 1, jnp.int32)

    def class_body(c, carry):
        def zero_chunk(k, _):
            hist_v[pl.ds(k * L, L)] = zvec
            return 0
        lax.fori_loop(0, NB2 // L, zero_chunk, 0)

        pltpu.sync_copy(p_hbm.at[pl.ds(0, NBATCH), c, pl.ds(base, CHUNK)], p_v)

        def batch_body(b, carry2):
            def vec_body(i, carry3):
                p = p_v[b, pl.ds(i * L, L)]
                labv = lab_v[b, pl.ds(i * L, L)]
                fg = labv == c
                e = jnp.where(fg, 1.0 - p, p)
                bkt = jnp.minimum((e * nbf).astype(jnp.int32), nbm1)
                idx = bkt + jnp.where(fg, fgoff, zoff)
                plsc.addupdate_scatter(hist_v, [idx], ones)
                return carry3
            return lax.fori_loop(0, CHUNK // L, vec_body, carry2)
        lax.fori_loop(0, NBATCH, batch_body, 0)

        pltpu.sync_copy(hist_v, out_hbm.at[wid, c])
        return carry
    lax.fori_loop(0, NUM_C, class_body, 0)


def _stage2_body(hist_hbm, rows_hbm, part_v, acc_v, row_v):
    cid = lax.axis_index("c")
    sid = lax.axis_index("s")
    iot = lax.iota(jnp.int32, L)
    zvec = jnp.zeros((L,), jnp.float32)

    @pl.when(cid == 0)
    def _core0():
        def do_row(r):
            @pl.when(r < NUM_C)
            def _active():
                c = jnp.minimum(r, NUM_C - 1)
                pltpu.sync_copy(hist_hbm.at[pl.ds(0, NW), c], part_v)

                def red_chunk(k, _):
                    def add_t(t, sv):
                        return sv + part_v[t, pl.ds(k * L, L)]
                    acc_v[pl.ds(k * L, L)] = lax.fori_loop(
                        0, NW, add_t, zvec)
                    return 0
                lax.fori_loop(0, NB2 // L, red_chunk, 0)

                def gsum(k, v):
                    return v + acc_v[pl.ds((NB // L + k) * L, L)]
                gts = jnp.sum(lax.fori_loop(0, NB // L, gsum, zvec))

                def scan_chunk(k, carry):
                    s_c, k_c, sj = carry
                    m = NB // L - 1 - k
                    f = acc_v[pl.ds((NB // L + m) * L, L)]
                    g = acc_v[pl.ds(m * L, L)]
                    n = f + g
                    fr = lax.rev(f, (0,))
                    nr = lax.rev(n, (0,))
                    s_vec = jnp.cumsum(fr) + s_c
                    k_vec = jnp.cumsum(nr) + k_c
                    union = gts + k_vec - s_vec
                    ratio = (gts - s_vec) / jnp.maximum(union, 1.0)
                    j = jnp.where(union > 0, 1.0 - ratio, 0.0)
                    return (jnp.max(s_vec), jnp.max(k_vec), sj + j)
                _, _, sjv = lax.fori_loop(
                    0, NB // L, scan_chunk,
                    (jnp.float32(0), jnp.float32(0), zvec))
                sumj = jnp.sum(sjv)
                loss_c = sumj * (1.0 / NB) - 0.5 / NB
                pres = jnp.where(gts > 0, 1.0, 0.0).astype(jnp.float32)
                row = jnp.where(iot == 0, loss_c * pres, 0.0) + \
                    jnp.where(iot == 1, pres, 0.0)
                row_v[...] = row
                pltpu.sync_copy(row_v, rows_hbm.at[r])

            @pl.when(r >= NUM_C)
            def _pad():
                row_v[...] = zvec
                pltpu.sync_copy(row_v, rows_hbm.at[r])

        do_row(sid)
        do_row(sid + NS)


def _stage3_body(rows_hbm, out_hbm, fin_v, row_v):
    cid = lax.axis_index("c")
    sid = lax.axis_index("s")
    iot = lax.iota(jnp.int32, L)
    zvec = jnp.zeros((L,), jnp.float32)

    @pl.when((cid == 0) & (sid == 0))
    def _final():
        pltpu.sync_copy(rows_hbm, fin_v)

        def addrow(r, v):
            return v + fin_v[r, pl.ds(0, L)]
        tot = lax.fori_loop(0, NW, addrow, zvec)
        lp_vec = jnp.take(tot, jnp.zeros((L,), jnp.int32))
        pp_vec = jnp.take(tot, jnp.ones((L,), jnp.int32))
        loss_vec = lp_vec / jnp.maximum(pp_vec, 1.0)
        row_v[...] = loss_vec
        pltpu.sync_copy(row_v, out_hbm)


def _build_calls():
    mesh = plsc.VectorSubcoreMesh(
        core_axis_name="c", subcore_axis_name="s",
        num_cores=NC, num_subcores=NS)

    params = pltpu.CompilerParams(needs_layout_passes=False)

    stage1 = functools.partial(
        pl.kernel, _stage1_body, mesh=mesh,
        compiler_params=params,
        out_type=jax.ShapeDtypeStruct((NW, NUM_C, NB2), jnp.float32),
        scratch_types=[
            pltpu.VMEM((NBATCH, CHUNK), jnp.int32),
            pltpu.VMEM((NBATCH, CHUNK), jnp.float32),
            pltpu.VMEM((NB2,), jnp.float32),
        ],
    )()

    stage2 = functools.partial(
        pl.kernel, _stage2_body, mesh=mesh,
        compiler_params=params,
        out_type=jax.ShapeDtypeStruct((NW, L), jnp.float32),
        scratch_types=[
            pltpu.VMEM((NW, NB2), jnp.float32),
            pltpu.VMEM((NB2,), jnp.float32),
            pltpu.VMEM((L,), jnp.float32),
        ],
    )()

    stage3 = functools.partial(
        pl.kernel, _stage3_body, mesh=mesh,
        compiler_params=params,
        out_type=jax.ShapeDtypeStruct((L,), jnp.float32),
        scratch_types=[
            pltpu.VMEM((NW, L), jnp.float32),
            pltpu.VMEM((L,), jnp.float32),
        ],
    )()
    return stage1, stage2, stage3


def kernel(probas, labels):
    b, c, h, w = probas.shape
    p3 = probas.reshape(b, c, h * w)
    lab2 = labels.reshape(b, h * w)
    stage1, stage2, stage3 = _build_calls()
    hist = stage1(p3, lab2)
    rows = stage2(hist)
    out16 = stage3(rows)
    return out16[0]


# speedup vs baseline: 125.0439x; 2.8269x over previous
import functools

import jax
import jax.numpy as jnp
from jax import lax
from jax.experimental import pallas as pl
from jax.experimental.pallas import tpu as pltpu
from jax.experimental.pallas import tpu_sc as plsc

NB = 1024
NB2 = 2 * NB
L = 16
NC = 2
NS = 16
NW = NC * NS
NUM_C = 21
PIX = 512 * 512
CHUNK = PIX // NW
NBATCH = 4


def _stage1_body(p_hbm, lab_hbm, out_hbm, lab_v, p_v, hist_v, in_sem):
    cid = lax.axis_index("c")
    sid = lax.axis_index("s")
    wid = sid * NC + cid
    base = wid * CHUNK

    pltpu.sync_copy(lab_hbm.at[pl.ds(0, NBATCH), pl.ds(base, CHUNK)], lab_v)

    ones = jnp.ones((L,), jnp.float32)
    zvec = jnp.zeros((L,), jnp.float32)
    nbf = jnp.float32(NB)
    nbm1 = jnp.full((L,), NB - 1, jnp.int32)
    mirr = jnp.full((L,), NB2 - 1, jnp.int32)

    def start_fetch(c, par):
        pltpu.make_async_copy(
            p_hbm.at[pl.ds(0, NBATCH), c, pl.ds(base, CHUNK)],
            p_v.at[par], in_sem).start()

    def wait_fetch(par):
        pltpu.make_async_copy(
            p_hbm.at[pl.ds(0, NBATCH), 0, pl.ds(base, CHUNK)],
            p_v.at[par], in_sem).wait()

    start_fetch(0, 0)

    def class_body(c, carry):
        par = lax.rem(c, 2)
        def zero_chunk(k, _):
            hist_v[pl.ds(k * L, L)] = zvec
            return 0
        lax.fori_loop(0, NB2 // L, zero_chunk, 0)

        wait_fetch(par)

        @pl.when(c + 1 < NUM_C)
        def _prefetch():
            start_fetch(c + 1, 1 - par)

        @plsc.parallel_loop(0, NBATCH * (CHUNK // L), unroll=8)
        def _vec(i):
            b = i // (CHUNK // L)
            j = i - b * (CHUNK // L)
            p = p_v[par, b, pl.ds(j * L, L)]
            labv = lab_v[b, pl.ds(j * L, L)]
            fg = labv == c
            bkt = jnp.minimum((p * nbf).astype(jnp.int32), nbm1)
            idx = jnp.where(fg, mirr - bkt, bkt)
            plsc.addupdate_scatter(hist_v, [idx], ones)

        pltpu.sync_copy(hist_v, out_hbm.at[wid, c])
        return carry
    lax.fori_loop(0, NUM_C, class_body, 0)


def _stage2_body(hist_hbm, rows_hbm, part_v, acc_v, row_v):
    cid = lax.axis_index("c")
    sid = lax.axis_index("s")
    wid = sid * NC + cid
    iot = lax.iota(jnp.int32, L)
    zvec = jnp.zeros((L,), jnp.float32)

    if True:
        def do_row(r):
            @pl.when(r < NUM_C)
            def _active():
                c = jnp.minimum(r, NUM_C - 1)
                pltpu.sync_copy(hist_hbm.at[pl.ds(0, NW), c], part_v)

                def red_chunk(k, _):
                    def add_t(t, sv):
                        return sv + part_v[t, pl.ds(k * L, L)]
                    acc_v[pl.ds(k * L, L)] = lax.fori_loop(
                        0, NW, add_t, zvec)
                    return 0
                lax.fori_loop(0, NB2 // L, red_chunk, 0)

                def gsum(k, v):
                    return v + acc_v[pl.ds((NB // L + k) * L, L)]
                gts = jnp.sum(lax.fori_loop(0, NB // L, gsum, zvec))

                def scan_chunk(k, carry):
                    s_c, k_c, sj = carry
                    m = NB // L - 1 - k
                    f = acc_v[pl.ds((NB // L + m) * L, L)]
                    g = acc_v[pl.ds(m * L, L)]
                    n = f + g
                    fr = lax.rev(f, (0,))
                    nr = lax.rev(n, (0,))
                    s_vec = jnp.cumsum(fr) + s_c
                    k_vec = jnp.cumsum(nr) + k_c
                    union = gts + k_vec - s_vec
                    ratio = (gts - s_vec) / jnp.maximum(union, 1.0)
                    j = jnp.where(union > 0, 1.0 - ratio, 0.0)
                    return (jnp.max(s_vec), jnp.max(k_vec), sj + j)
                _, _, sjv = lax.fori_loop(
                    0, NB // L, scan_chunk,
                    (jnp.float32(0), jnp.float32(0), zvec))
                sumj = jnp.sum(sjv)
                loss_c = sumj * (1.0 / NB) - 0.5 / NB
                pres = jnp.where(gts > 0, 1.0, 0.0).astype(jnp.float32)
                row = jnp.where(iot == 0, loss_c * pres, 0.0) + \
                    jnp.where(iot == 1, pres, 0.0)
                row_v[...] = row
                pltpu.sync_copy(row_v, rows_hbm.at[r])

            @pl.when(r >= NUM_C)
            def _pad():
                row_v[...] = zvec
                pltpu.sync_copy(row_v, rows_hbm.at[r])

        do_row(wid)


def _stage3_body(rows_hbm, out_hbm, fin_v, row_v):
    cid = lax.axis_index("c")
    sid = lax.axis_index("s")
    iot = lax.iota(jnp.int32, L)
    zvec = jnp.zeros((L,), jnp.float32)

    @pl.when((cid == 0) & (sid == 0))
    def _final():
        pltpu.sync_copy(rows_hbm, fin_v)

        def addrow(r, v):
            return v + fin_v[r, pl.ds(0, L)]
        tot = lax.fori_loop(0, NW, addrow, zvec)
        lp_vec = jnp.take(tot, jnp.zeros((L,), jnp.int32))
        pp_vec = jnp.take(tot, jnp.ones((L,), jnp.int32))
        loss_vec = lp_vec / jnp.maximum(pp_vec, 1.0)
        row_v[...] = loss_vec
        pltpu.sync_copy(row_v, out_hbm)


def _build_calls():
    mesh = plsc.VectorSubcoreMesh(
        core_axis_name="c", subcore_axis_name="s",
        num_cores=NC, num_subcores=NS)

    params = pltpu.CompilerParams(needs_layout_passes=False)

    stage1 = functools.partial(
        pl.kernel, _stage1_body, mesh=mesh,
        compiler_params=params,
        out_type=jax.ShapeDtypeStruct((NW, NUM_C, NB2), jnp.float32),
        scratch_types=[
            pltpu.VMEM((NBATCH, CHUNK), jnp.int32),
            pltpu.VMEM((2, NBATCH, CHUNK), jnp.float32),
            pltpu.VMEM((NB2,), jnp.float32),
            pltpu.SemaphoreType.DMA,
        ],
    )()

    stage2 = functools.partial(
        pl.kernel, _stage2_body, mesh=mesh,
        compiler_params=params,
        out_type=jax.ShapeDtypeStruct((NW, L), jnp.float32),
        scratch_types=[
            pltpu.VMEM((NW, NB2), jnp.float32),
            pltpu.VMEM((NB2,), jnp.float32),
            pltpu.VMEM((L,), jnp.float32),
        ],
    )()

    stage3 = functools.partial(
        pl.kernel, _stage3_body, mesh=mesh,
        compiler_params=params,
        out_type=jax.ShapeDtypeStruct((L,), jnp.float32),
        scratch_types=[
            pltpu.VMEM((NW, L), jnp.float32),
            pltpu.VMEM((L,), jnp.float32),
        ],
    )()
    return stage1, stage2, stage3


def kernel(probas, labels):
    b, c, h, w = probas.shape
    p3 = probas.reshape(b, c, h * w)
    lab2 = labels.reshape(b, h * w)
    stage1, stage2, stage3 = _build_calls()
    hist = stage1(p3, lab2)
    rows = stage2(hist)
    out16 = stage3(rows)
    return out16[0]


# speedup vs baseline: 136.8825x; 1.0947x over previous
import functools

import jax
import jax.numpy as jnp
from jax import lax
from jax.experimental import pallas as pl
from jax.experimental.pallas import tpu as pltpu
from jax.experimental.pallas import tpu_sc as plsc

NB = 1024
NB2 = 2 * NB
L = 16
NC = 2
NS = 16
NW = NC * NS
NUM_C = 21
PIX = 512 * 512
CHUNK = PIX // NW
NBATCH = 4


def _stage1_body(p_hbm, lab_hbm, out_hbm, lab_v, p_v, hist_v, in_sem):
    cid = lax.axis_index("c")
    sid = lax.axis_index("s")
    wid = sid * NC + cid
    base = wid * CHUNK

    pltpu.sync_copy(lab_hbm.at[pl.ds(0, NBATCH), pl.ds(base, CHUNK)], lab_v)

    ones = jnp.ones((L,), jnp.float32)
    zvec = jnp.zeros((L,), jnp.float32)
    nbf = jnp.float32(NB)
    nbm1 = jnp.full((L,), NB - 1, jnp.int32)
    mirr = jnp.full((L,), NB2 - 1, jnp.int32)

    def start_fetch(c, par):
        pltpu.make_async_copy(
            p_hbm.at[pl.ds(0, NBATCH), c, pl.ds(base, CHUNK)],
            p_v.at[par], in_sem).start()

    def wait_fetch(par):
        pltpu.make_async_copy(
            p_hbm.at[pl.ds(0, NBATCH), 0, pl.ds(base, CHUNK)],
            p_v.at[par], in_sem).wait()

    start_fetch(0, 0)

    def class_body(c, carry):
        par = lax.rem(c, 2)
        def zero_chunk(k, _):
            hist_v[pl.ds(k * L, L)] = zvec
            return 0
        lax.fori_loop(0, NB2 // L, zero_chunk, 0)

        wait_fetch(par)

        @pl.when(c + 1 < NUM_C)
        def _prefetch():
            start_fetch(c + 1, 1 - par)

        @plsc.parallel_loop(0, NBATCH * (CHUNK // L), unroll=8)
        def _vec(i):
            b = i // (CHUNK // L)
            j = i - b * (CHUNK // L)
            p = p_v[par, b, pl.ds(j * L, L)]
            labv = lab_v[b, pl.ds(j * L, L)]
            fg = labv == c
            bkt = jnp.minimum((p * nbf).astype(jnp.int32), nbm1)
            idx = jnp.where(fg, mirr - bkt, bkt)
            plsc.addupdate_scatter(hist_v, [idx], ones)

        pltpu.sync_copy(hist_v, out_hbm.at[wid, c])
        return carry
    lax.fori_loop(0, NUM_C, class_body, 0)


def _stage23_tc_body(hist_ref, out_ref):
    acc = jnp.sum(hist_ref[...], axis=0)
    n_fg = acc[:, NB:]
    n_tot = acc[:, :NB] + n_fg
    row_i = lax.broadcasted_iota(jnp.int32, (NB, NB), 0)
    col_i = lax.broadcasted_iota(jnp.int32, (NB, NB), 1)
    m_tri = (row_i >= col_i).astype(jnp.float32)
    s_suf = jnp.dot(n_fg, m_tri, preferred_element_type=jnp.float32)
    k_suf = jnp.dot(n_tot, m_tri, preferred_element_type=jnp.float32)
    gts = s_suf[:, :1]
    union = gts + k_suf - s_suf
    ratio = (gts - s_suf) / jnp.maximum(union, 1.0)
    j = jnp.where(union > 0, 1.0 - ratio, 0.0)
    loss_c = jnp.sum(j, axis=1) * (1.0 / NB) - 0.5 / NB
    pres = jnp.where(gts[:, 0] > 0, 1.0, 0.0)
    loss = jnp.sum(loss_c * pres) / jnp.maximum(jnp.sum(pres), 1.0)
    out_ref[...] = jnp.broadcast_to(loss, (1, 1))


def _build_calls():
    mesh = plsc.VectorSubcoreMesh(
        core_axis_name="c", subcore_axis_name="s",
        num_cores=NC, num_subcores=NS)

    params = pltpu.CompilerParams(needs_layout_passes=False)

    stage1 = functools.partial(
        pl.kernel, _stage1_body, mesh=mesh,
        compiler_params=params,
        out_type=jax.ShapeDtypeStruct((NW, NUM_C, NB2), jnp.float32),
        scratch_types=[
            pltpu.VMEM((NBATCH, CHUNK), jnp.int32),
            pltpu.VMEM((2, NBATCH, CHUNK), jnp.float32),
            pltpu.VMEM((NB2,), jnp.float32),
            pltpu.SemaphoreType.DMA,
        ],
    )()

    stage23 = pl.pallas_call(
        _stage23_tc_body,
        out_shape=jax.ShapeDtypeStruct((1, 1), jnp.float32),
    )
    return stage1, stage23


def kernel(probas, labels):
    b, c, h, w = probas.shape
    p3 = probas.reshape(b, c, h * w)
    lab2 = labels.reshape(b, h * w)
    stage1, stage23 = _build_calls()
    hist = stage1(p3, lab2)
    out = stage23(hist)
    return out[0, 0]
